# baseline (speedup 1.0000x reference)
import jax
import jax.numpy as jnp
from jax.experimental import pallas as pl
from jax.experimental.pallas import tpu as pltpu

N_AGENTS = 20
BATCH = 2500
INV_NF = 12
HID = 64
N_LAYERS = 2
DEG = float(N_AGENTS - 1)

B_TILE = 32
BATCH_PAD = 2560


def _silu(x):
    return x * (1.0 / (1.0 + jnp.exp(-x)))


def _dot(a, b):
    return jnp.dot(a, b, preferred_element_type=jnp.float32)


def _egnn_body(x_ref, loc_ref, act_ref, *refs):
    out_ref = refs[-1]
    w = [r[...] for r in refs[:-1]]
    B, A, H = B_TILE, N_AGENTS, HID
    BA = B * A

    k = iter(range(len(w)))
    Wemb, bemb = w[next(k)], w[next(k)]

    h = _dot(x_ref[...], Wemb) + bemb
    loc = loc_ref[...]
    v = act_ref[...]

    locr = loc.reshape(B, A, 2)
    cd0 = locr[:, :, None, :] - locr[:, None, :, :]
    ea = jnp.sum(cd0 * cd0, axis=-1, keepdims=True).reshape(BA * A, 1)

    for _ in range(N_LAYERS):
        (W1r, W1c, Wre, b1, W2, b2, Wc1, bc1, Wc2,
         Wn1h, Wn1a, bn1, Wn2, bn2, Wv1, bv1, Wv2, bv2) = (
            w[next(k)] for _ in range(18))

        locr = loc.reshape(B, A, 2)
        cd = locr[:, :, None, :] - locr[:, None, :, :]
        radial = jnp.sum(cd * cd, axis=-1, keepdims=True)
        cdn = cd / (jnp.sqrt(radial) + 1.0)

        hr = _dot(h, W1r)
        hc = _dot(h, W1c) + b1
        rflat = radial.reshape(BA * A, 1)
        rterm = _dot(jnp.concatenate([rflat, ea], axis=1), Wre)
        pre = (hr.reshape(B, A, 1, H) + hc.reshape(B, 1, A, H)
               + rterm.reshape(B, A, A, H))
        e1 = _silu(pre.reshape(BA * A, H))
        m = _silu(_dot(e1, W2) + b2)

        c1 = _silu(_dot(m, Wc1) + bc1)
        s = _dot(c1, Wc2).reshape(B, A, A, 1)
        agg = jnp.sum(cdn * s, axis=2) / DEG

        velf = _dot(_silu(_dot(h, Wv1) + bv1), Wv2) + bv2
        v = velf * v + agg.reshape(BA, 2)
        loc = loc + v

        m_diag = _silu(_dot(_silu(hr + hc), W2) + b2)
        nag = jnp.sum(m.reshape(B, A, A, H), axis=2).reshape(BA, H) - m_diag
        n1 = _silu(_dot(h, Wn1h) + _dot(nag, Wn1a) + bn1)
        h = h + _dot(n1, Wn2) + bn2

    Wq, bq = w[next(k)], w[next(k)]
    q = _dot(jnp.tanh(h), Wq) + bq
    out_ref[...] = jnp.sum(q.reshape(B, A, 1), axis=1) / float(A)


def kernel(cent_obs, actions, params, rows, cols):
    del rows, cols
    N = BATCH * N_AGENTS
    pad_n = (BATCH_PAD - BATCH) * N_AGENTS

    x = cent_obs.reshape(N, INV_NF + 4)
    loc0 = x[:, INV_NF:INV_NF + 2]
    xp = jnp.pad(x, ((0, pad_n), (0, 0)))
    locp = jnp.pad(loc0, ((0, pad_n), (0, 0)))
    actp = jnp.pad(actions, ((0, pad_n), (0, 0)))

    Wemb, bemb = params["emb"]
    wlist = [jnp.pad(Wemb, ((0, 4), (0, 0))), bemb.reshape(1, HID)]
    for layer in params["layers"]:
        W1, b1 = layer["edge1"]
        W2, b2 = layer["edge2"]
        Wn1, bn1 = layer["node1"]
        Wn2, bn2 = layer["node2"]
        Wc1, bc1 = layer["coord1"]
        (Wc2,) = layer["coord2"]
        Wv1, bv1 = layer["vel1"]
        Wv2, bv2 = layer["vel2"]
        wlist += [
            W1[:HID], W1[HID:2 * HID], W1[2 * HID:], b1.reshape(1, HID),
            W2, b2.reshape(1, HID),
            Wc1, bc1.reshape(1, HID), Wc2,
            Wn1[:HID], Wn1[HID:], bn1.reshape(1, HID),
            Wn2, bn2.reshape(1, HID),
            Wv1, bv1.reshape(1, HID), Wv2, bv2.reshape(1, 1),
        ]
    Wq, bq = params["critic"]
    wlist += [Wq, bq.reshape(1, 1)]

    grid = (BATCH_PAD // B_TILE,)
    row_spec = lambda width: pl.BlockSpec((B_TILE * N_AGENTS, width),
                                          lambda i: (i, 0))
    w_specs = [pl.BlockSpec(wl.shape, lambda i: (0, 0)) for wl in wlist]

    out = pl.pallas_call(
        _egnn_body,
        grid=grid,
        in_specs=[row_spec(INV_NF + 4), row_spec(2), row_spec(2)] + w_specs,
        out_specs=pl.BlockSpec((B_TILE, 1), lambda i: (i, 0)),
        out_shape=jax.ShapeDtypeStruct((BATCH_PAD, 1), jnp.float32),
        compiler_params=pltpu.CompilerParams(
            dimension_semantics=("parallel",)),
    )(xp, locp, actp, *wlist)
    return out[:BATCH]


# speedup vs baseline: 1.5590x; 1.5590x over previous
import jax
import jax.numpy as jnp
from jax.experimental import pallas as pl
from jax.experimental.pallas import tpu as pltpu

N_AGENTS = 20
BATCH = 2500
INV_NF = 12
HID = 64
N_LAYERS = 2
DEG = float(N_AGENTS - 1)

B_TILE = 32
BATCH_PAD = 2560


def _silu(x):
    return x * (1.0 / (1.0 + jnp.exp(-x)))


def _dot(a, b):
    return jnp.dot(a, b, preferred_element_type=jnp.float32)


def _egnn_body(x_ref, loc_ref, act_ref, *refs):
    out_ref = refs[-1]
    w = [r[...] for r in refs[:-1]]
    B, A, H = B_TILE, N_AGENTS, HID
    AB = A * B

    k = iter(range(len(w)))
    Wemb, bemb = w[next(k)], w[next(k)]

    h = _dot(x_ref[...].reshape(AB, INV_NF + 4), Wemb) + bemb
    loc = loc_ref[...].reshape(AB, 2)
    v = act_ref[...].reshape(AB, 2)

    locr = loc.reshape(A, B, 2)
    cd0 = locr[:, None] - locr[None, :]
    ea = jnp.sum(cd0 * cd0, axis=-1, keepdims=True)

    for _ in range(N_LAYERS):
        (W1r, W1c, wr, we, b1, W2, b2, Wc1, bc1, Wc2,
         Wn1h, Wn1a, bn1, Wn2, bn2, Wv1, bv1, Wv2, bv2) = (
            w[next(k)] for _ in range(19))

        locr = loc.reshape(A, B, 2)
        cd = locr[:, None] - locr[None, :]
        radial = jnp.sum(cd * cd, axis=-1, keepdims=True)
        cdn = cd / (jnp.sqrt(radial) + 1.0)

        hr = _dot(h, W1r)
        hc = _dot(h, W1c) + b1
        pre = (hr.reshape(A, 1, B, H) + hc.reshape(1, A, B, H)
               + radial * wr + ea * we)
        e1 = _silu(pre.reshape(AB * A, H))
        m = _silu(_dot(e1, W2) + b2)

        c1 = _silu(_dot(m, Wc1) + bc1)
        s = _dot(c1, Wc2).reshape(A, A, B, 1)
        agg = jnp.sum(cdn * s, axis=1).reshape(AB, 2) / DEG

        velf = _dot(_silu(_dot(h, Wv1) + bv1), Wv2) + bv2
        v = velf * v + agg
        loc = loc + v

        m_diag = _silu(_dot(_silu(hr + hc), W2) + b2)
        nag = jnp.sum(m.reshape(A, A, B, H), axis=1).reshape(AB, H) - m_diag
        n1 = _silu(_dot(h, Wn1h) + _dot(nag, Wn1a) + bn1)
        h = h + _dot(n1, Wn2) + bn2

    Wq, bq = w[next(k)], w[next(k)]
    q = _dot(jnp.tanh(h), Wq) + bq
    out_ref[...] = jnp.sum(q.reshape(A, B, 1), axis=0) / float(A)


def kernel(cent_obs, actions, params, rows, cols):
    del rows, cols
    pad_b = BATCH_PAD - BATCH
    F = INV_NF + 4

    def to_agent_major(arr, f):
        a = arr.reshape(BATCH, N_AGENTS, f).transpose(1, 0, 2)
        return jnp.pad(a, ((0, 0), (0, pad_b), (0, 0)))

    x = cent_obs.reshape(BATCH * N_AGENTS, F)
    xp = to_agent_major(x, F)
    locp = to_agent_major(x[:, INV_NF:INV_NF + 2], 2)
    actp = to_agent_major(actions, 2)

    Wemb, bemb = params["emb"]
    wlist = [jnp.pad(Wemb, ((0, 4), (0, 0))), bemb.reshape(1, HID)]
    for layer in params["layers"]:
        W1, b1 = layer["edge1"]
        W2, b2 = layer["edge2"]
        Wn1, bn1 = layer["node1"]
        Wn2, bn2 = layer["node2"]
        Wc1, bc1 = layer["coord1"]
        (Wc2,) = layer["coord2"]
        Wv1, bv1 = layer["vel1"]
        Wv2, bv2 = layer["vel2"]
        wlist += [
            W1[:HID], W1[HID:2 * HID], W1[2 * HID:2 * HID + 1],
            W1[2 * HID + 1:], b1.reshape(1, HID),
            W2, b2.reshape(1, HID),
            Wc1, bc1.reshape(1, HID), Wc2,
            Wn1[:HID], Wn1[HID:], bn1.reshape(1, HID),
            Wn2, bn2.reshape(1, HID),
            Wv1, bv1.reshape(1, HID), Wv2, bv2.reshape(1, 1),
        ]
    Wq, bq = params["critic"]
    wlist += [Wq, bq.reshape(1, 1)]

    grid = (BATCH_PAD // B_TILE,)
    row_spec = lambda width: pl.BlockSpec((N_AGENTS, B_TILE, width),
                                          lambda i: (0, i, 0))
    w_specs = [pl.BlockSpec(wl.shape, lambda i: (0, 0)) for wl in wlist]

    out = pl.pallas_call(
        _egnn_body,
        grid=grid,
        in_specs=[row_spec(F), row_spec(2), row_spec(2)] + w_specs,
        out_specs=pl.BlockSpec((B_TILE, 1), lambda i: (i, 0)),
        out_shape=jax.ShapeDtypeStruct((BATCH_PAD, 1), jnp.float32),
        compiler_params=pltpu.CompilerParams(
            dimension_semantics=("parallel",)),
    )(xp, locp, actp, *wlist)
    return out[:BATCH]


# speedup vs baseline: 2.3124x; 1.4833x over previous
import jax
import jax.numpy as jnp
import numpy as np
from jax.experimental import pallas as pl
from jax.experimental.pallas import tpu as pltpu

N_AGENTS = 20
BATCH = 2500
INV_NF = 12
HID = 64
N_LAYERS = 2
DEG = float(N_AGENTS - 1)

B2_TILE = 16
PAIRS = BATCH // 2
PAIRS_PAD = 1280


def _silu(x):
    return x * (1.0 / (1.0 + jnp.exp(-x)))


def _dot(a, b):
    return jnp.dot(a, b, preferred_element_type=jnp.float32)


def _egnn_body(x_ref, loc_ref, act_ref, *refs):
    out_ref = refs[-1]
    w = [r[...] for r in refs[:-1]]
    A, B2, H2 = N_AGENTS, B2_TILE, 2 * HID
    AB = A * B2
    E = A * A * B2

    k = iter(range(len(w)))
    S42, S24, Wemb, bemb = (w[next(k)] for _ in range(4))

    h = _dot(x_ref[...].reshape(AB, 2 * (INV_NF + 4)), Wemb) + bemb
    loc = loc_ref[...].reshape(AB, 4)
    v = act_ref[...].reshape(AB, 4)

    locr = loc.reshape(A, B2, 4)
    cd0 = locr[:, None] - locr[None, :]
    ea2 = _dot((cd0 * cd0).reshape(E, 4), S42)

    for _ in range(N_LAYERS):
        (W1r, W1c, Wre_r, Wre_e, b1, W2, b2, Wc1, bc1, Wc2,
         Wn1h, Wn1a, bn1, Wn2, bn2, Wv1, bv1, Wv2, bv2) = (
            w[next(k)] for _ in range(19))

        locr = loc.reshape(A, B2, 4)
        cd = locr[:, None] - locr[None, :]
        radial2 = _dot((cd * cd).reshape(E, 4), S42)
        inv2 = 1.0 / (jnp.sqrt(radial2) + 1.0)
        cdn = cd * _dot(inv2, S24).reshape(A, A, B2, 4)

        hr = _dot(h, W1r)
        hc = _dot(h, W1c) + b1
        rterm = (_dot(radial2, Wre_r) + _dot(ea2, Wre_e)).reshape(A, A, B2, H2)
        pre = hr.reshape(A, 1, B2, H2) + hc.reshape(1, A, B2, H2) + rterm
        e1 = _silu(pre.reshape(E, H2))
        m = _silu(_dot(e1, W2) + b2)

        c1 = _silu(_dot(m, Wc1) + bc1)
        s4 = _dot(_dot(c1, Wc2), S24).reshape(A, A, B2, 4)
        agg = jnp.sum(cdn * s4, axis=1).reshape(AB, 4)

        velf = _dot(_dot(_silu(_dot(h, Wv1) + bv1), Wv2) + bv2, S24)
        v = velf * v + agg
        loc = loc + v

        m_diag = _silu(_dot(_silu(hr + hc), W2) + b2)
        nag = jnp.sum(m.reshape(A, A, B2, H2), axis=1).reshape(AB, H2) - m_diag
        n1 = _silu(_dot(h, Wn1h) + _dot(nag, Wn1a) + bn1)
        h = h + _dot(n1, Wn2) + bn2

    Wq, bq = w[next(k)], w[next(k)]
    q = _dot(jnp.tanh(h), Wq) + bq
    out_ref[...] = jnp.sum(q.reshape(A, B2, 2), axis=0)


def _bd(wm):
    z = jnp.zeros_like(wm)
    return jnp.concatenate(
        [jnp.concatenate([wm, z], axis=1), jnp.concatenate([z, wm], axis=1)],
        axis=0)


def _dup(b):
    r = b.reshape(1, -1)
    return jnp.concatenate([r, r], axis=1)


def kernel(cent_obs, actions, params, rows, cols):
    del rows, cols
    F = INV_NF + 4
    A = N_AGENTS

    def pack(arr, f):
        a = arr.reshape(PAIRS, 2, A, f).transpose(2, 0, 1, 3)
        a = a.reshape(A, PAIRS, 2 * f)
        return jnp.pad(a, ((0, 0), (0, PAIRS_PAD - PAIRS), (0, 0)))

    x = cent_obs.reshape(BATCH * A, F)
    xp = pack(x, F)
    locp = pack(x[:, INV_NF:INV_NF + 2], 2)
    actp = pack(actions, 2)

    s42 = jnp.asarray(np.array([[1, 0], [1, 0], [0, 1], [0, 1]], np.float32))
    s24 = jnp.asarray(np.array([[1, 1, 0, 0], [0, 0, 1, 1]], np.float32))

    Wemb, bemb = params["emb"]
    wlist = [s42, s24, _bd(jnp.pad(Wemb, ((0, 4), (0, 0)))), _dup(bemb)]
    for layer in params["layers"]:
        W1, b1 = layer["edge1"]
        W2, b2 = layer["edge2"]
        Wn1, bn1 = layer["node1"]
        Wn2, bn2 = layer["node2"]
        Wc1, bc1 = layer["coord1"]
        (Wc2,) = layer["coord2"]
        Wv1, bv1 = layer["vel1"]
        Wv2, bv2 = layer["vel2"]
        wlist += [
            _bd(W1[:HID]), _bd(W1[HID:2 * HID]),
            _bd(W1[2 * HID:2 * HID + 1]), _bd(W1[2 * HID + 1:]), _dup(b1),
            _bd(W2), _dup(b2),
            _bd(Wc1), _dup(bc1), _bd(Wc2 / DEG),
            _bd(Wn1[:HID]), _bd(Wn1[HID:]), _dup(bn1),
            _bd(Wn2), _dup(bn2),
            _bd(Wv1), _dup(bv1), _bd(Wv2), _dup(bv2),
        ]
    Wq, bq = params["critic"]
    wlist += [_bd(Wq / A), _dup(bq / A)]

    grid = (PAIRS_PAD // B2_TILE,)
    row_spec = lambda width: pl.BlockSpec((A, B2_TILE, width),
                                          lambda i: (0, i, 0))
    w_specs = [pl.BlockSpec(wl.shape, lambda i: (0, 0)) for wl in wlist]

    out = pl.pallas_call(
        _egnn_body,
        grid=grid,
        in_specs=[row_spec(2 * F), row_spec(4), row_spec(4)] + w_specs,
        out_specs=pl.BlockSpec((B2_TILE, 2), lambda i: (i, 0)),
        out_shape=jax.ShapeDtypeStruct((PAIRS_PAD, 2), jnp.float32),
        compiler_params=pltpu.CompilerParams(
            dimension_semantics=("parallel",)),
    )(xp, locp, actp, *wlist)
    return out.reshape(-1, 1)[:BATCH]
